# Initial kernel scaffold; baseline (speedup 1.0000x reference)
import jax
import jax.numpy as jnp
from jax.experimental import pallas as pl


def kernel(x, expert_W, expert_b, router_W, router_b):
    raise NotImplementedError("write your pallas kernel here")


# speedup vs baseline: 1.2502x; 1.2502x over previous
import functools

import jax
import jax.numpy as jnp
from jax.experimental import pallas as pl
from jax.experimental.pallas import tpu as pltpu

N_EXPERTS = 16
TOP_K = 2
D = 1024
N_TOKENS = 8192
TBLK = 512


def _moe_dense_body(x_ref, w_ref, b_ref, rw_ref, rb_ref, out_ref):
    e = pl.program_id(1)
    xb = x_ref[...]
    logits = jax.lax.dot_general(
        xb, rw_ref[...], (((1,), (1,)), ((), ())),
        preferred_element_type=jnp.float32) + rb_ref[0][None, :]
    probs = jax.nn.softmax(logits, axis=-1)
    m0 = jnp.max(probs, axis=-1, keepdims=True)
    i0 = jnp.argmax(probs, axis=-1)
    masked = jnp.where(jax.lax.broadcasted_iota(jnp.int32, probs.shape, 1)
                       == i0[:, None], -jnp.inf, probs)
    m1 = jnp.max(masked, axis=-1, keepdims=True)
    i1 = jnp.argmax(masked, axis=-1)
    w_e = jnp.where(i0[:, None] == e, m0, 0.0) + jnp.where(i1[:, None] == e, m1, 0.0)
    y = jax.lax.dot_general(
        xb, w_ref[0], (((1,), (1,)), ((), ())),
        preferred_element_type=jnp.float32) + b_ref[0]

    @pl.when(e == 0)
    def _():
        out_ref[...] = jnp.zeros_like(out_ref)

    out_ref[...] += w_e * y


def kernel(x, expert_W, expert_b, router_W, router_b):
    grid = (N_TOKENS // TBLK, N_EXPERTS)
    return pl.pallas_call(
        _moe_dense_body,
        grid=grid,
        in_specs=[
            pl.BlockSpec((TBLK, D), lambda t, e: (t, 0)),
            pl.BlockSpec((1, D, D), lambda t, e: (e, 0, 0)),
            pl.BlockSpec((1, 1, D), lambda t, e: (e, 0, 0)),
            pl.BlockSpec((N_EXPERTS, D), lambda t, e: (0, 0)),
            pl.BlockSpec((1, N_EXPERTS), lambda t, e: (0, 0)),
        ],
        out_specs=pl.BlockSpec((TBLK, D), lambda t, e: (t, 0)),
        out_shape=jax.ShapeDtypeStruct((N_TOKENS, D), jnp.float32),
    )(x, expert_W, expert_b.reshape(N_EXPERTS, 1, D), router_W,
      router_b.reshape(1, N_EXPERTS))


# speedup vs baseline: 1.8838x; 1.5068x over previous
import functools

import jax
import jax.numpy as jnp
from jax import lax
from jax.experimental import pallas as pl
from jax.experimental.pallas import tpu as pltpu
from jax.experimental.pallas import tpu_sc as plsc

NE = 16
D = 1024
NT = 8192
NA = 2 * NT
TBLK = 512
BLK = 256
NBLK = NA // BLK + NE
PAD = NBLK * BLK

NC, NS = 2, 16
NW = NC * NS
TPW = NT // NW
CH_D = 32
CH_C = 32


def _router_body(x_ref, rw_ref, rb_ref, i0_ref, i1_ref, g0_ref, g1_ref):
    xb = x_ref[...]
    logits = lax.dot_general(xb, rw_ref[...], (((1,), (1,)), ((), ())),
                             preferred_element_type=jnp.float32) + rb_ref[0][None, :]
    probs = jax.nn.softmax(logits, axis=-1)
    cols = lax.broadcasted_iota(jnp.int32, probs.shape, 1)
    m0 = jnp.max(probs, axis=-1, keepdims=True)
    i0 = jnp.argmax(probs, axis=-1)
    masked = jnp.where(cols == i0[:, None], -jnp.inf, probs)
    m1 = jnp.max(masked, axis=-1, keepdims=True)
    i1 = jnp.argmax(masked, axis=-1)
    i0_ref[...] = i0.reshape(1, 1, TBLK)
    i1_ref[...] = i1.reshape(1, 1, TBLK)
    g0_ref[...] = jnp.broadcast_to(m0, (TBLK, NE))
    g1_ref[...] = jnp.broadcast_to(m1, (TBLK, NE))


def _router(x, router_W, router_b):
    nblk = NT // TBLK
    return pl.pallas_call(
        _router_body,
        grid=(nblk,),
        in_specs=[
            pl.BlockSpec((TBLK, D), lambda t: (t, 0)),
            pl.BlockSpec((NE, D), lambda t: (0, 0)),
            pl.BlockSpec((1, NE), lambda t: (0, 0)),
        ],
        out_specs=[
            pl.BlockSpec((1, 1, TBLK), lambda t: (t, 0, 0)),
            pl.BlockSpec((1, 1, TBLK), lambda t: (t, 0, 0)),
            pl.BlockSpec((TBLK, NE), lambda t: (t, 0)),
            pl.BlockSpec((TBLK, NE), lambda t: (t, 0)),
        ],
        out_shape=[
            jax.ShapeDtypeStruct((nblk, 1, TBLK), jnp.int32),
            jax.ShapeDtypeStruct((nblk, 1, TBLK), jnp.int32),
            jax.ShapeDtypeStruct((NT, NE), jnp.float32),
            jax.ShapeDtypeStruct((NT, NE), jnp.float32),
        ],
    )(x, router_W, router_b.reshape(1, NE))


_G = 128


def _book_body(e0_ref, e1_ref, p0_ref, p1_ref, be_ref):
    ea = jnp.concatenate([e0_ref[...], e1_ref[...]], axis=0)
    rows = lax.broadcasted_iota(jnp.int32, (_G, _G), 0)
    colsq = lax.broadcasted_iota(jnp.int32, (_G, _G), 1)
    U = (rows <= colsq).astype(jnp.float32)
    Lx = (colsq < rows).astype(jnp.float32)

    onehots, totals = [], []
    for e in range(NE):
        ohf = (ea == e).astype(jnp.float32)
        C = lax.dot_general(ohf, U, (((1,), (0,)), ((), ())),
                            preferred_element_type=jnp.float32)
        S = C[:, _G - 1:_G]
        P = lax.dot_general(Lx, S, (((1,), (0,)), ((), ())),
                            preferred_element_type=jnp.float32)
        rank = (P + C).astype(jnp.int32) - 1
        onehots.append((ohf.astype(jnp.int32), rank))
        totals.append(jnp.sum(S).astype(jnp.int32))

    pos = jnp.zeros((_G, _G), jnp.int32)
    start_blocks = []
    sbk = jnp.int32(0)
    for e in range(NE):
        start_blocks.append(sbk)
        oh, rank = onehots[e]
        pos = pos + oh * (sbk * BLK + rank)
        sbk = sbk + (totals[e] + (BLK - 1)) // BLK

    p0_ref[...] = pos[:_G // 2]
    p1_ref[...] = pos[_G // 2:]

    biota = lax.broadcasted_iota(jnp.int32, (8, 128), 1)
    be = jnp.zeros((8, 128), jnp.int32)
    for e in range(NE):
        be = be + (biota >= start_blocks[e]).astype(jnp.int32)
    be_ref[...] = be - 1


def _bookkeeping(e0_2d, e1_2d):
    return pl.pallas_call(
        _book_body,
        out_shape=[
            jax.ShapeDtypeStruct((_G // 2, _G), jnp.int32),
            jax.ShapeDtypeStruct((_G // 2, _G), jnp.int32),
            jax.ShapeDtypeStruct((8, 128), jnp.int32),
        ],
    )(e0_2d, e1_2d)


def _dispatch(x, p0, p1):
    mesh = plsc.VectorSubcoreMesh(core_axis_name="c", subcore_axis_name="s")

    @functools.partial(
        pl.kernel, mesh=mesh,
        out_type=jax.ShapeDtypeStruct((PAD, D), jnp.float32),
        scratch_types=[
            pltpu.VMEM((CH_D, D), jnp.float32),
            pltpu.VMEM((CH_D,), jnp.int32),
            pltpu.VMEM((CH_D,), jnp.int32),
            pltpu.SemaphoreType.DMA,
        ],
    )
    def body(x_hbm, p0_hbm, p1_hbm, xs_hbm, xbuf, i0v, i1v, sem):
        wid = lax.axis_index("s") * NC + lax.axis_index("c")
        t0 = wid * TPW

        def chunk(i, carry):
            base = t0 + i * CH_D
            pltpu.sync_copy(x_hbm.at[pl.ds(base, CH_D)], xbuf)
            pltpu.sync_copy(p0_hbm.at[pl.ds(base, CH_D)], i0v)
            pltpu.sync_copy(p1_hbm.at[pl.ds(base, CH_D)], i1v)
            pltpu.async_copy(xbuf, xs_hbm.at[i0v], sem).wait()
            pltpu.async_copy(xbuf, xs_hbm.at[i1v], sem).wait()
            return carry

        lax.fori_loop(0, TPW // CH_D, chunk, 0)

    return body(x, p0, p1)


def _gmm_body(be_ref, xs_ref, w_ref, b_ref, y_ref):
    del be_ref
    y_ref[...] = lax.dot_general(
        xs_ref[...], w_ref[0], (((1,), (1,)), ((), ())),
        preferred_element_type=jnp.float32) + b_ref[0]


def _grouped_matmul(be, xs, expert_W, expert_b):
    grid_spec = pltpu.PrefetchScalarGridSpec(
        num_scalar_prefetch=1,
        grid=(NBLK,),
        in_specs=[
            pl.BlockSpec((BLK, D), lambda i, be_s: (i, 0)),
            pl.BlockSpec((1, D, D), lambda i, be_s: (be_s[i], 0, 0)),
            pl.BlockSpec((1, 1, D), lambda i, be_s: (be_s[i], 0, 0)),
        ],
        out_specs=pl.BlockSpec((BLK, D), lambda i, be_s: (i, 0)),
    )
    return pl.pallas_call(
        _gmm_body,
        grid_spec=grid_spec,
        out_shape=jax.ShapeDtypeStruct((PAD, D), jnp.float32),
    )(be, xs, expert_W, expert_b.reshape(NE, 1, D))


def _combine(ys, p0, p1, g0b, g1b):
    mesh = plsc.VectorSubcoreMesh(core_axis_name="c", subcore_axis_name="s")

    @functools.partial(
        pl.kernel, mesh=mesh,
        out_type=jax.ShapeDtypeStruct((NT, D), jnp.float32),
        scratch_types=[
            pltpu.VMEM((CH_C, D), jnp.float32),
            pltpu.VMEM((CH_C, D), jnp.float32),
            pltpu.VMEM((CH_C, D), jnp.float32),
            pltpu.VMEM((CH_C,), jnp.int32),
            pltpu.VMEM((CH_C,), jnp.int32),
            pltpu.VMEM((CH_C, NE), jnp.float32),
            pltpu.VMEM((CH_C, NE), jnp.float32),
            pltpu.SemaphoreType.DMA,
            pltpu.SemaphoreType.DMA,
        ],
    )
    def body(y_hbm, p0_hbm, p1_hbm, g0_hbm, g1_hbm, out_hbm,
             rows0, rows1, ob, i0v, i1v, g0v, g1v, sem0, sem1):
        wid = lax.axis_index("s") * NC + lax.axis_index("c")
        t0 = wid * TPW

        def chunk(i, carry):
            base = t0 + i * CH_C
            pltpu.sync_copy(p0_hbm.at[pl.ds(base, CH_C)], i0v)
            pltpu.sync_copy(p1_hbm.at[pl.ds(base, CH_C)], i1v)
            pltpu.sync_copy(g0_hbm.at[pl.ds(base, CH_C)], g0v)
            pltpu.sync_copy(g1_hbm.at[pl.ds(base, CH_C)], g1v)
            c0 = pltpu.async_copy(y_hbm.at[i0v], rows0, sem0)
            c1 = pltpu.async_copy(y_hbm.at[i1v], rows1, sem1)
            c0.wait()
            c1.wait()
            for r in range(CH_C):
                g0r = g0v[r]
                g1r = g1v[r]

                def col(c, carry2):
                    off = c * NS
                    ob[r, pl.ds(off, NS)] = (g0r * rows0[r, pl.ds(off, NS)]
                                             + g1r * rows1[r, pl.ds(off, NS)])
                    return carry2

                lax.fori_loop(0, D // NS, col, 0)
            pltpu.sync_copy(ob, out_hbm.at[pl.ds(base, CH_C)])
            return carry

        lax.fori_loop(0, TPW // CH_C, chunk, 0)

    return body(ys, p0, p1, g0b, g1b)


def kernel(x, expert_W, expert_b, router_W, router_b):
    i0, i1, g0b, g1b = _router(x, router_W, router_b)
    e0_2d = i0.reshape(NT).reshape(_G // 2, _G)
    e1_2d = i1.reshape(NT).reshape(_G // 2, _G)
    p0_2d, p1_2d, be2d = _bookkeeping(e0_2d, e1_2d)
    p0 = p0_2d.reshape(NT)
    p1 = p1_2d.reshape(NT)
    be = be2d.reshape(1024)[:NBLK]
    xs = _dispatch(x, p0, p1)
    ys = _grouped_matmul(be, xs, expert_W, expert_b)
    return _combine(ys, p0, p1, g0b, g1b)


# speedup vs baseline: 1.9773x; 1.0496x over previous
import functools

import jax
import jax.numpy as jnp
from jax import lax
from jax.experimental import pallas as pl
from jax.experimental.pallas import tpu as pltpu
from jax.experimental.pallas import tpu_sc as plsc

NE = 16
D = 1024
NT = 8192
NA = 2 * NT
TBLK = 512
BLK = 256
NBLK = NA // BLK + NE
PAD = NBLK * BLK

NC, NS = 2, 16
NW = NC * NS
TPW = NT // NW
CH = 16
NCH = TPW // CH
CHE = 8
NCHE = TPW // CHE


def _router_body(x_ref, rw_ref, rb_ref, i0_ref, i1_ref, g0_ref, g1_ref):
    xb = x_ref[...]
    logits = lax.dot_general(xb, rw_ref[...], (((1,), (1,)), ((), ())),
                             preferred_element_type=jnp.float32) + rb_ref[0][None, :]
    probs = jax.nn.softmax(logits, axis=-1)
    cols = lax.broadcasted_iota(jnp.int32, probs.shape, 1)
    m0 = jnp.max(probs, axis=-1, keepdims=True)
    i0 = jnp.argmax(probs, axis=-1)
    masked = jnp.where(cols == i0[:, None], -jnp.inf, probs)
    m1 = jnp.max(masked, axis=-1, keepdims=True)
    i1 = jnp.argmax(masked, axis=-1)
    i0_ref[...] = i0.reshape(1, 1, TBLK)
    i1_ref[...] = i1.reshape(1, 1, TBLK)
    g0_ref[...] = jnp.broadcast_to(m0, (TBLK, NE))
    g1_ref[...] = jnp.broadcast_to(m1, (TBLK, NE))


def _router(x, router_W, router_b):
    nblk = NT // TBLK
    return pl.pallas_call(
        _router_body,
        grid=(nblk,),
        in_specs=[
            pl.BlockSpec((TBLK, D), lambda t: (t, 0)),
            pl.BlockSpec((NE, D), lambda t: (0, 0)),
            pl.BlockSpec((1, NE), lambda t: (0, 0)),
        ],
        out_specs=[
            pl.BlockSpec((1, 1, TBLK), lambda t: (t, 0, 0)),
            pl.BlockSpec((1, 1, TBLK), lambda t: (t, 0, 0)),
            pl.BlockSpec((TBLK, NE), lambda t: (t, 0)),
            pl.BlockSpec((TBLK, NE), lambda t: (t, 0)),
        ],
        out_shape=[
            jax.ShapeDtypeStruct((nblk, 1, TBLK), jnp.int32),
            jax.ShapeDtypeStruct((nblk, 1, TBLK), jnp.int32),
            jax.ShapeDtypeStruct((NT, NE), jnp.float32),
            jax.ShapeDtypeStruct((NT, NE), jnp.float32),
        ],
    )(x, router_W, router_b.reshape(1, NE))


_G = 128


def _book_body(e0_ref, e1_ref, p0_ref, p1_ref, be_ref):
    ea = jnp.concatenate([e0_ref[...], e1_ref[...]], axis=0)
    rows = lax.broadcasted_iota(jnp.int32, (_G, _G), 0)
    colsq = lax.broadcasted_iota(jnp.int32, (_G, _G), 1)
    U = (rows <= colsq).astype(jnp.float32)
    Lx = (colsq < rows).astype(jnp.float32)

    onehots, totals = [], []
    for e in range(NE):
        ohf = (ea == e).astype(jnp.float32)
        C = lax.dot_general(ohf, U, (((1,), (0,)), ((), ())),
                            preferred_element_type=jnp.float32)
        S = C[:, _G - 1:_G]
        P = lax.dot_general(Lx, S, (((1,), (0,)), ((), ())),
                            preferred_element_type=jnp.float32)
        rank = (P + C).astype(jnp.int32) - 1
        onehots.append((ohf.astype(jnp.int32), rank))
        totals.append(jnp.sum(S).astype(jnp.int32))

    pos = jnp.zeros((_G, _G), jnp.int32)
    start_blocks = []
    sbk = jnp.int32(0)
    for e in range(NE):
        start_blocks.append(sbk)
        oh, rank = onehots[e]
        pos = pos + oh * (sbk * BLK + rank)
        sbk = sbk + (totals[e] + (BLK - 1)) // BLK

    p0_ref[...] = pos[:_G // 2]
    p1_ref[...] = pos[_G // 2:]

    biota = lax.broadcasted_iota(jnp.int32, (8, 128), 1)
    be = jnp.zeros((8, 128), jnp.int32)
    for e in range(NE):
        be = be + (biota >= start_blocks[e]).astype(jnp.int32)
    be_ref[...] = be - 1


def _bookkeeping(e0_2d, e1_2d):
    return pl.pallas_call(
        _book_body,
        out_shape=[
            jax.ShapeDtypeStruct((_G // 2, _G), jnp.int32),
            jax.ShapeDtypeStruct((_G // 2, _G), jnp.int32),
            jax.ShapeDtypeStruct((8, 128), jnp.int32),
        ],
    )(e0_2d, e1_2d)


def _dispatch(x, p0r, p1r):
    mesh = plsc.VectorSubcoreMesh(core_axis_name="c", subcore_axis_name="s")

    @functools.partial(
        pl.kernel, mesh=mesh,
        out_type=jax.ShapeDtypeStruct((PAD, D), jnp.float32),
        scratch_types=[
            pltpu.VMEM((CH, D), jnp.float32),
            pltpu.VMEM((CH, D), jnp.float32),
            pltpu.VMEM((NCH, CH), jnp.int32),
            pltpu.VMEM((NCH, CH), jnp.int32),
            pltpu.SemaphoreType.DMA,
            pltpu.SemaphoreType.DMA,
            pltpu.SemaphoreType.DMA,
        ],
    )
    def body(x_hbm, p0_hbm, p1_hbm, xs_hbm,
             xb0, xb1, i0all, i1all, ldsem0, ldsem1, scsem):
        wid = lax.axis_index("s") * NC + lax.axis_index("c")
        t0 = wid * TPW
        r0 = wid * NCH
        pltpu.sync_copy(p0_hbm.at[pl.ds(r0, NCH)], i0all)
        pltpu.sync_copy(p1_hbm.at[pl.ds(r0, NCH)], i1all)

        xbufs = (xb0, xb1)
        ldsems = (ldsem0, ldsem1)
        loads = [None, None]
        loads[0] = pltpu.async_copy(x_hbm.at[pl.ds(t0, CH)], xb0, ldsem0)
        pending = []
        for j in range(NCH):
            b = j % 2
            loads[b].wait()
            if pending:
                for c in pending.pop(0):
                    c.wait()
            if j + 1 < NCH:
                loads[1 - b] = pltpu.async_copy(
                    x_hbm.at[pl.ds(t0 + (j + 1) * CH, CH)],
                    xbufs[1 - b], ldsems[1 - b])
            pending.append([
                pltpu.async_copy(xbufs[b], xs_hbm.at[i0all.at[j]], scsem),
                pltpu.async_copy(xbufs[b], xs_hbm.at[i1all.at[j]], scsem),
            ])
        for grp in pending:
            for c in grp:
                c.wait()

    return body(x, p0r, p1r)


def _gmm_body(be_ref, xs_ref, w_ref, b_ref, y_ref):
    del be_ref
    y_ref[...] = lax.dot_general(
        xs_ref[...], w_ref[0], (((1,), (1,)), ((), ())),
        preferred_element_type=jnp.float32) + b_ref[0]


def _grouped_matmul(be, xs, expert_W, expert_b):
    grid_spec = pltpu.PrefetchScalarGridSpec(
        num_scalar_prefetch=1,
        grid=(NBLK,),
        in_specs=[
            pl.BlockSpec((BLK, D), lambda i, be_s: (i, 0)),
            pl.BlockSpec((1, D, D), lambda i, be_s: (be_s[i], 0, 0)),
            pl.BlockSpec((1, 1, D), lambda i, be_s: (be_s[i], 0, 0)),
        ],
        out_specs=pl.BlockSpec((BLK, D), lambda i, be_s: (i, 0)),
    )
    return pl.pallas_call(
        _gmm_body,
        grid_spec=grid_spec,
        out_shape=jax.ShapeDtypeStruct((PAD, D), jnp.float32),
    )(be, xs, expert_W, expert_b.reshape(NE, 1, D))


def _combine(ys, pcat, g0b, g1b):
    mesh = plsc.VectorSubcoreMesh(core_axis_name="c", subcore_axis_name="s")

    @functools.partial(
        pl.kernel, mesh=mesh,
        out_type=jax.ShapeDtypeStruct((NT, D), jnp.float32),
        scratch_types=[
            pltpu.VMEM((2 * CHE, D), jnp.float32),
            pltpu.VMEM((2 * CHE, D), jnp.float32),
            pltpu.VMEM((CHE, D), jnp.float32),
            pltpu.VMEM((NCHE, 2 * CHE), jnp.int32),
            pltpu.VMEM((TPW, NE), jnp.float32),
            pltpu.VMEM((TPW, NE), jnp.float32),
            pltpu.SemaphoreType.DMA,
            pltpu.SemaphoreType.DMA,
            pltpu.SemaphoreType.DMA,
        ],
    )
    def body(y_hbm, pc_hbm, g0_hbm, g1_hbm, out_hbm, rb0, rb1, ob,
             pcall, g0all, g1all, gsem0, gsem1, osem):
        wid = lax.axis_index("s") * NC + lax.axis_index("c")
        t0 = wid * TPW
        r0 = wid * NCHE
        pltpu.sync_copy(pc_hbm.at[pl.ds(r0, NCHE)], pcall)
        pltpu.sync_copy(g0_hbm.at[pl.ds(t0, TPW)], g0all)
        pltpu.sync_copy(g1_hbm.at[pl.ds(t0, TPW)], g1all)

        rbufs = (rb0, rb1)
        gsems = (gsem0, gsem1)
        gathers = [None, None]
        owrite = [None]
        gathers[0] = pltpu.async_copy(y_hbm.at[pcall.at[0]], rb0, gsem0)
        for j in range(NCHE):
            b = j % 2
            gathers[b].wait()
            if j + 1 < NCHE:
                gathers[1 - b] = pltpu.async_copy(
                    y_hbm.at[pcall.at[j + 1]], rbufs[1 - b], gsems[1 - b])
            if owrite[0] is not None:
                owrite[0].wait()
            rb = rbufs[b]
            goff = j * CHE

            def row(r, carry):
                g0v = g0all[goff + r]
                g1v = g1all[goff + r]

                def col(c, carry2):
                    sl = pl.ds(c * 16, 16)
                    ob[r, sl] = g0v * rb[r, sl] + g1v * rb[r + CHE, sl]
                    return carry2

                lax.fori_loop(0, D // 16, col, 0)
                return carry

            lax.fori_loop(0, CHE, row, 0)
            owrite[0] = pltpu.async_copy(
                ob, out_hbm.at[pl.ds(t0 + j * CHE, CHE)], osem)
        owrite[0].wait()

    return body(ys, pcat, g0b, g1b)


def kernel(x, expert_W, expert_b, router_W, router_b):
    i0, i1, g0b, g1b = _router(x, router_W, router_b)
    e0_2d = i0.reshape(NT).reshape(_G // 2, _G)
    e1_2d = i1.reshape(NT).reshape(_G // 2, _G)
    p0_2d, p1_2d, be2d = _bookkeeping(e0_2d, e1_2d)
    p0 = p0_2d.reshape(NT)
    p1 = p1_2d.reshape(NT)
    be = be2d.reshape(1024)[:NBLK]
    p0r = p0.reshape(NT // CH, CH)
    p1r = p1.reshape(NT // CH, CH)
    xs = _dispatch(x, p0r, p1r)
    ys = _grouped_matmul(be, xs, expert_W, expert_b)
    pcat = jnp.concatenate(
        [p0.reshape(NT // CHE, CHE), p1.reshape(NT // CHE, CHE)], axis=1)
    return _combine(ys, pcat, g0b, g1b)
